# baseline (speedup 1.0000x reference)
import functools

import jax
import jax.numpy as jnp
from jax import lax
from jax.experimental import pallas as pl
from jax.experimental.pallas import tpu as pltpu
from jax.experimental.pallas import tpu_sc as plsc

N_NODES = 100000
BATCH = 10000
S = 16
D = 128

NC = 2
NS = 16
NW = NC * NS
BP = 10240
BPW = BP // NW
CROWS = 8
NCHUNK = BPW // CROWS
RED = 6240
REDTAIL = N_NODES - RED * NS


def _sc_body(nodes2d, neigh2d, feat, rc, rowv, clumv, out,
             nidx2, gidx2, rcn_v, rcnb_v, scale_v, redbuf, tailbuf,
             pbuf, allbuf, obuf, gbuf0, gbuf1, shared, sem0, semg):
    cid = lax.axis_index("c")
    sid = lax.axis_index("s")
    wid = sid * NC + cid
    base = wid * BPW

    neg = jnp.full((16,), -3.0e38, jnp.float32)

    def _reduce_slice(src):
        pltpu.sync_copy(src.at[pl.ds(sid * RED, RED)], redbuf)
        pltpu.sync_copy(src.at[pl.ds(NS * RED, REDTAIL)], tailbuf)

        def body(i, m):
            for k in range(10):
                m = jnp.maximum(m, redbuf[pl.ds(i * 160 + k * 16, 16)])
            return m

        m = lax.fori_loop(0, RED // 160, body, neg)
        for k in range(REDTAIL // 16):
            m = jnp.maximum(m, tailbuf[pl.ds(k * 16, 16)])
        return m

    mrow = _reduce_slice(rowv)
    mclum = _reduce_slice(clumv)

    pbuf[0, :] = mrow
    pbuf[1, :] = mclum
    pltpu.sync_copy(pbuf, shared.at[sid])
    plsc.subcore_barrier()
    pltpu.sync_copy(shared, allbuf)
    for t in range(NS):
        mrow = jnp.maximum(mrow, allbuf[t, 0, :])
        mclum = jnp.maximum(mclum, allbuf[t, 1, :])

    il = lax.iota(jnp.int32, 16)

    def _lane_max(v):
        for sh in (8, 4, 2, 1):
            tailbuf[pl.ds(0, 16)] = v
            v = jnp.maximum(v, plsc.load_gather(tailbuf, [il ^ sh]))
        return v

    inv_b = 1.0 / _lane_max(mrow)
    inv_a = 1.0 / _lane_max(mclum)

    pltpu.sync_copy(nodes2d.at[pl.ds(wid * 8, 8)], nidx2)
    pltpu.sync_copy(neigh2d.at[pl.ds(wid * NCHUNK, NCHUNK)], gidx2)

    descs = []
    for t in range(8):
        dsc = pltpu.make_async_copy(
            rc.at[nidx2.at[t]], rcn_v.at[pl.ds(t * 40, 40)], sem0)
        dsc.start()
        descs.append(dsc)
    for dd in descs:
        dd.wait()

    zero16 = jnp.zeros((16,), jnp.int32)
    one16 = zero16 + 1

    def dw_body(g, carry):
        d0 = pltpu.make_async_copy(
            rc.at[gidx2.at[2 * g]], rcnb_v.at[pl.ds(0, 128)], sem0)
        d1 = pltpu.make_async_copy(
            rc.at[gidx2.at[2 * g + 1]], rcnb_v.at[pl.ds(128, 128)], sem0)
        d0.start()
        d1.start()
        d0.wait()
        d1.wait()
        rbase = g * 16 + il
        rown = plsc.load_gather(rcn_v, [rbase, zero16]) * inv_b
        clumn = plsc.load_gather(rcn_v, [rbase, one16]) * inv_a
        srow = jnp.zeros((16,), jnp.float32)
        sclum = jnp.zeros((16,), jnp.float32)
        nb = il * S
        for j in range(S):
            srow = srow + plsc.load_gather(rcnb_v, [nb + j, zero16])
            sclum = sclum + plsc.load_gather(rcnb_v, [nb + j, one16])
        row_sum = srow * (1.0 / S) * inv_b
        clum_sum = sclum * (1.0 / S) * inv_a
        dr = row_sum - rown
        dc = clum_sum - clumn
        d2 = dr * dr + dc * dc + 1e-12
        dw = 1.0 / (1.0 + jnp.exp(-1.0 / d2))
        scale_v[pl.ds(g * 16, 16)] = dw * (1.0 / S)
        return carry

    lax.fori_loop(0, BPW // 16, dw_body, 0)

    gb = (gbuf0, gbuf1)

    pltpu.make_async_copy(feat.at[gidx2.at[0]], gb[0], semg).start()

    def chunk_compute(gbuf, c):
        def row_body(r, carry):
            accs = [gbuf[r * S, pl.ds(k * 16, 16)] for k in range(D // 16)]
            for j in range(1, S):
                for k in range(D // 16):
                    accs[k] = accs[k] + gbuf[r * S + j, pl.ds(k * 16, 16)]
            widx = jnp.zeros((16,), jnp.int32) + (c * CROWS + r)
            w2 = 2.0 * plsc.load_gather(scale_v, [widx])
            for k in range(D // 16):
                e = jnp.exp(w2 * accs[k])
                obuf[r, pl.ds(k * 16, 16)] = (e - 1.0) / (e + 1.0)
            return carry

        lax.fori_loop(0, CROWS, row_body, 0)

    def outer(t, carry):
        for p in range(2):
            c = t * 2 + p
            pltpu.make_async_copy(feat.at[gidx2.at[c]], gb[p], semg).wait()

            @pl.when(c + 1 < NCHUNK)
            def _fire():
                pltpu.make_async_copy(
                    feat.at[gidx2.at[c + 1]], gb[1 - p], semg).start()

            chunk_compute(gb[p], c)
            s = base + c * CROWS

            @pl.when(s < BATCH)
            def _store():
                pltpu.sync_copy(obuf, out.at[pl.ds(s, CROWS)])
        return carry

    lax.fori_loop(0, NCHUNK // 2, outer, 0)


@jax.jit
def kernel(nodes, neigh_idx, features, row, clum):
    nodes_p = jnp.pad(nodes, (0, BP - BATCH)).reshape(BP // 40, 40)
    neigh_p = jnp.pad(neigh_idx.reshape(-1), (0, (BP - BATCH) * S))
    neigh2d = neigh_p.reshape(BP * S // 128, 128)
    rc = jnp.concatenate(
        [row[:, None], clum[:, None],
         jnp.zeros((N_NODES, 14), jnp.float32)], axis=1)

    mesh = plsc.VectorSubcoreMesh(core_axis_name="c", subcore_axis_name="s")
    f = functools.partial(
        pl.kernel,
        out_type=jax.ShapeDtypeStruct((BATCH, D), jnp.float32),
        mesh=mesh,
        compiler_params=pltpu.CompilerParams(
            needs_layout_passes=False, use_tc_tiling_on_sc=False),
        scratch_types=[
            pltpu.VMEM((8, 40), jnp.int32),
            pltpu.VMEM((NCHUNK, 128), jnp.int32),
            pltpu.VMEM((BPW, 16), jnp.float32),
            pltpu.VMEM((256, 16), jnp.float32),
            pltpu.VMEM((BPW,), jnp.float32),
            pltpu.VMEM((RED,), jnp.float32),
            pltpu.VMEM((REDTAIL,), jnp.float32),
            pltpu.VMEM((2, 16), jnp.float32),
            pltpu.VMEM((NS, 2, 16), jnp.float32),
            pltpu.VMEM((CROWS, D), jnp.float32),
            pltpu.VMEM((128, D), jnp.float32),
            pltpu.VMEM((128, D), jnp.float32),
            pltpu.VMEM_SHARED((NS, 2, 16), jnp.float32),
            pltpu.SemaphoreType.DMA,
            pltpu.SemaphoreType.DMA,
        ],
    )(_sc_body)
    return f(nodes_p, neigh2d, features, rc, row, clum)


# speedup vs baseline: 1.0569x; 1.0569x over previous
import functools

import jax
import jax.numpy as jnp
from jax import lax
from jax.experimental import pallas as pl
from jax.experimental.pallas import tpu as pltpu
from jax.experimental.pallas import tpu_sc as plsc

N_NODES = 100000
BATCH = 10000
S = 16
D = 128

NC = 2
NS = 16
NW = NC * NS
BP = 10240
BPW = BP // NW
CROWS = 8
NCHUNK = BPW // CROWS
RED = 6240
REDTAIL = N_NODES - RED * NS
NGRP = BPW // 16


def _sc_body(nodes2d, neigh2d, feat, rc, rowv, clumv, out,
             nidx2, gidx2, rcn_v, rcnb_v, scale_v, redbuf, tailbuf,
             pbuf, allbuf, obuf, gbuf0, gbuf1, gbuf2, gbuf3, shared,
             semi, semn, semc0, semc1, semf0, semf1, semf2, semf3,
             semo0, semo1):
    cid = lax.axis_index("c")
    sid = lax.axis_index("s")
    wid = sid * NC + cid
    base = wid * BPW

    d_nidx = pltpu.make_async_copy(nodes2d.at[pl.ds(wid * 8, 8)], nidx2, semi)
    d_gidx = pltpu.make_async_copy(
        neigh2d.at[pl.ds(wid * NCHUNK, NCHUNK)], gidx2, semi)
    d_nidx.start()
    d_gidx.start()

    neg = jnp.full((16,), -3.0e38, jnp.float32)

    def _reduce_slice(src):
        pltpu.sync_copy(src.at[pl.ds(sid * RED, RED)], redbuf)
        pltpu.sync_copy(src.at[pl.ds(NS * RED, REDTAIL)], tailbuf)

        def body(i, m):
            for k in range(10):
                m = jnp.maximum(m, redbuf[pl.ds(i * 160 + k * 16, 16)])
            return m

        m = lax.fori_loop(0, RED // 160, body, neg)
        for k in range(REDTAIL // 16):
            m = jnp.maximum(m, tailbuf[pl.ds(k * 16, 16)])
        return m

    mrow = _reduce_slice(rowv)
    mclum = _reduce_slice(clumv)

    pbuf[0, :] = mrow
    pbuf[1, :] = mclum
    pltpu.sync_copy(pbuf, shared.at[sid])
    plsc.subcore_barrier()
    pltpu.sync_copy(shared, allbuf)
    for t in range(NS):
        mrow = jnp.maximum(mrow, allbuf[t, 0, :])
        mclum = jnp.maximum(mclum, allbuf[t, 1, :])

    il = lax.iota(jnp.int32, 16)

    def _lane_max(v):
        for sh in (8, 4, 2, 1):
            tailbuf[pl.ds(0, 16)] = v
            v = jnp.maximum(v, plsc.load_gather(tailbuf, [il ^ sh]))
        return v

    inv_b = 1.0 / _lane_max(mrow)
    inv_a = 1.0 / _lane_max(mclum)

    d_nidx.wait()
    d_gidx.wait()

    gb = (gbuf0, gbuf1, gbuf2, gbuf3)
    semf = (semf0, semf1, semf2, semf3)

    def feat_desc(c, p):
        return pltpu.make_async_copy(feat.at[gidx2.at[c]], gb[p], semf[p])

    for c in range(3):
        feat_desc(c, c).start()

    ndescs = []
    for t in range(8):
        dsc = pltpu.make_async_copy(
            rc.at[nidx2.at[t]], rcn_v.at[pl.ds(t * 40, 40)], semn)
        dsc.start()
        ndescs.append(dsc)

    zero16 = jnp.zeros((16,), jnp.int32)
    one16 = zero16 + 1
    semc = (semc0, semc1)

    def coord_descs(g, p):
        return (
            pltpu.make_async_copy(
                rc.at[gidx2.at[2 * g]], rcnb_v.at[p, pl.ds(0, 128)], semc[p]),
            pltpu.make_async_copy(
                rc.at[gidx2.at[2 * g + 1]], rcnb_v.at[p, pl.ds(128, 128)],
                semc[p]),
        )

    for g in range(2):
        for dsc in coord_descs(g, g):
            dsc.start()

    for dd in ndescs:
        dd.wait()

    def dw_group(g, p):
        for dsc in coord_descs(g, p):
            dsc.wait()

        @pl.when(g + 2 < NGRP)
        def _fire():
            for dsc in coord_descs(g + 2, p):
                dsc.start()

        rbase = g * 16 + il
        rown = plsc.load_gather(rcn_v, [rbase, zero16]) * inv_b
        clumn = plsc.load_gather(rcn_v, [rbase, one16]) * inv_a
        srow = jnp.zeros((16,), jnp.float32)
        sclum = jnp.zeros((16,), jnp.float32)
        nb = il * S
        for j in range(S):
            srow = srow + plsc.load_gather(rcnb_v.at[p], [nb + j, zero16])
            sclum = sclum + plsc.load_gather(rcnb_v.at[p], [nb + j, one16])
        row_sum = srow * (1.0 / S) * inv_b
        clum_sum = sclum * (1.0 / S) * inv_a
        dr = row_sum - rown
        dc = clum_sum - clumn
        d2 = dr * dr + dc * dc + 1e-12
        dw = 1.0 / (1.0 + jnp.exp(-1.0 / d2))
        scale_v[pl.ds(g * 16, 16)] = dw * (1.0 / S)

    def dw_outer(t, carry):
        for p in range(2):
            dw_group(t * 2 + p, p)
        return carry

    lax.fori_loop(0, NGRP // 2, dw_outer, 0)

    semo = (semo0, semo1)

    def store_desc(c, po):
        s = base + c * CROWS
        return pltpu.make_async_copy(
            obuf.at[po], out.at[pl.ds(s, CROWS)], semo[po])

    def chunk_compute(gbuf, c, po):
        def row_body(r, carry):
            accs = [gbuf[r * S, pl.ds(k * 16, 16)] for k in range(D // 16)]
            for j in range(1, S):
                for k in range(D // 16):
                    accs[k] = accs[k] + gbuf[r * S + j, pl.ds(k * 16, 16)]
            widx = jnp.zeros((16,), jnp.int32) + (c * CROWS + r)
            w2 = 2.0 * plsc.load_gather(scale_v, [widx])
            for k in range(D // 16):
                e = jnp.exp(w2 * accs[k])
                obuf[po, r, pl.ds(k * 16, 16)] = (e - 1.0) / (e + 1.0)
            return carry

        lax.fori_loop(0, CROWS, row_body, 0)

    def outer(t, carry):
        for p in range(4):
            c = t * 4 + p
            po = p % 2
            feat_desc(c, p).wait()

            @pl.when(c + 3 < NCHUNK)
            def _fire():
                feat_desc(c + 3, (p + 3) % 4).start()

            @pl.when(base + (c - 2) * CROWS < BATCH)
            def _drain():
                @pl.when(c >= 2)
                def _():
                    store_desc(c - 2, po).wait()

            chunk_compute(gb[p], c, po)

            @pl.when(base + c * CROWS < BATCH)
            def _store():
                store_desc(c, po).start()
        return carry

    lax.fori_loop(0, NCHUNK // 4, outer, 0)

    for c in (NCHUNK - 2, NCHUNK - 1):
        @pl.when(base + c * CROWS < BATCH)
        def _drain_tail(c=c):
            store_desc(c, c % 2).wait()


@jax.jit
def kernel(nodes, neigh_idx, features, row, clum):
    nodes_p = jnp.pad(nodes, (0, BP - BATCH)).reshape(BP // 40, 40)
    neigh_p = jnp.pad(neigh_idx.reshape(-1), (0, (BP - BATCH) * S))
    neigh2d = neigh_p.reshape(BP * S // 128, 128)
    rc = jnp.concatenate(
        [row[:, None], clum[:, None],
         jnp.zeros((N_NODES, 14), jnp.float32)], axis=1)

    mesh = plsc.VectorSubcoreMesh(core_axis_name="c", subcore_axis_name="s")
    f = functools.partial(
        pl.kernel,
        out_type=jax.ShapeDtypeStruct((BATCH, D), jnp.float32),
        mesh=mesh,
        compiler_params=pltpu.CompilerParams(
            needs_layout_passes=False, use_tc_tiling_on_sc=False),
        scratch_types=[
            pltpu.VMEM((8, 40), jnp.int32),
            pltpu.VMEM((NCHUNK, 128), jnp.int32),
            pltpu.VMEM((BPW, 16), jnp.float32),
            pltpu.VMEM((2, 256, 16), jnp.float32),
            pltpu.VMEM((BPW,), jnp.float32),
            pltpu.VMEM((RED,), jnp.float32),
            pltpu.VMEM((REDTAIL,), jnp.float32),
            pltpu.VMEM((2, 16), jnp.float32),
            pltpu.VMEM((NS, 2, 16), jnp.float32),
            pltpu.VMEM((2, CROWS, D), jnp.float32),
            pltpu.VMEM((128, D), jnp.float32),
            pltpu.VMEM((128, D), jnp.float32),
            pltpu.VMEM((128, D), jnp.float32),
            pltpu.VMEM((128, D), jnp.float32),
            pltpu.VMEM_SHARED((NS, 2, 16), jnp.float32),
            pltpu.SemaphoreType.DMA,
            pltpu.SemaphoreType.DMA,
            pltpu.SemaphoreType.DMA,
            pltpu.SemaphoreType.DMA,
            pltpu.SemaphoreType.DMA,
            pltpu.SemaphoreType.DMA,
            pltpu.SemaphoreType.DMA,
            pltpu.SemaphoreType.DMA,
            pltpu.SemaphoreType.DMA,
            pltpu.SemaphoreType.DMA,
        ],
    )(_sc_body)
    return f(nodes_p, neigh2d, features, rc, row, clum)


# speedup vs baseline: 1.0626x; 1.0053x over previous
import functools

import jax
import jax.numpy as jnp
from jax import lax
from jax.experimental import pallas as pl
from jax.experimental.pallas import tpu as pltpu
from jax.experimental.pallas import tpu_sc as plsc

N_NODES = 100000
BATCH = 10000
S = 16
D = 128

NC = 2
NS = 16
NW = NC * NS
BP = 10240
BPW = BP // NW
CROWS = 8
NCHUNK = BPW // CROWS
RED = 6240
REDTAIL = N_NODES - RED * NS
NGRP = BPW // 16


def _sc_body(nodes2d, neigh2d, feat, rc, rowv, clumv, out,
             nidx2, gidx2, rcn_v, rcnb_v, scale_v, redbuf, tailbuf,
             pbuf, allbuf, obuf, gbuf0, gbuf1, gbuf2, gbuf3, shared,
             semi, semn, semc0, semc1, semf0, semf1, semf2, semf3,
             semo0, semo1):
    cid = lax.axis_index("c")
    sid = lax.axis_index("s")
    wid = sid * NC + cid
    base = wid * BPW

    d_nidx = pltpu.make_async_copy(nodes2d.at[pl.ds(wid * 8, 8)], nidx2, semi)
    d_gidx = pltpu.make_async_copy(
        neigh2d.at[pl.ds(wid * NCHUNK, NCHUNK)], gidx2, semi)
    d_nidx.start()
    d_gidx.start()

    scope_maxred = jax.named_scope("maxred")
    scope_maxred.__enter__()
    neg = jnp.full((16,), -3.0e38, jnp.float32)

    def _reduce_slice(src):
        pltpu.sync_copy(src.at[pl.ds(sid * RED, RED)], redbuf)
        pltpu.sync_copy(src.at[pl.ds(NS * RED, REDTAIL)], tailbuf)

        def body(i, m):
            for k in range(10):
                m = jnp.maximum(m, redbuf[pl.ds(i * 160 + k * 16, 16)])
            return m

        m = lax.fori_loop(0, RED // 160, body, neg)
        for k in range(REDTAIL // 16):
            m = jnp.maximum(m, tailbuf[pl.ds(k * 16, 16)])
        return m

    mrow = _reduce_slice(rowv)
    mclum = _reduce_slice(clumv)

    pbuf[0, :] = mrow
    pbuf[1, :] = mclum
    pltpu.sync_copy(pbuf, shared.at[sid])
    plsc.subcore_barrier()
    pltpu.sync_copy(shared, allbuf)
    for t in range(NS):
        mrow = jnp.maximum(mrow, allbuf[t, 0, :])
        mclum = jnp.maximum(mclum, allbuf[t, 1, :])

    il = lax.iota(jnp.int32, 16)

    def _lane_max(v):
        for sh in (8, 4, 2, 1):
            tailbuf[pl.ds(0, 16)] = v
            v = jnp.maximum(v, plsc.load_gather(tailbuf, [il ^ sh]))
        return v

    inv_b = 1.0 / _lane_max(mrow)
    inv_a = 1.0 / _lane_max(mclum)
    scope_maxred.__exit__(None, None, None)

    scope_stage = jax.named_scope("idxwait")
    scope_stage.__enter__()
    d_nidx.wait()
    d_gidx.wait()
    scope_stage.__exit__(None, None, None)

    gb = (gbuf0, gbuf1, gbuf2, gbuf3)
    semf = (semf0, semf1, semf2, semf3)

    def feat_desc(c, p):
        return pltpu.make_async_copy(feat.at[gidx2.at[c]], gb[p], semf[p])

    for c in range(3):
        feat_desc(c, c).start()

    ndescs = []
    for t in range(8):
        dsc = pltpu.make_async_copy(
            rc.at[nidx2.at[t]], rcn_v.at[pl.ds(t * 40, 40)], semn)
        dsc.start()
        ndescs.append(dsc)

    zero16 = jnp.zeros((16,), jnp.int32)
    one16 = zero16 + 1
    semc = (semc0, semc1)

    def coord_descs(g, p):
        return (
            pltpu.make_async_copy(
                rc.at[gidx2.at[2 * g]], rcnb_v.at[p, pl.ds(0, 128)], semc[p]),
            pltpu.make_async_copy(
                rc.at[gidx2.at[2 * g + 1]], rcnb_v.at[p, pl.ds(128, 128)],
                semc[p]),
        )

    for g in range(2):
        for dsc in coord_descs(g, g):
            dsc.start()

    for dd in ndescs:
        dd.wait()

    def dw_group(g, p):
        for dsc in coord_descs(g, p):
            dsc.wait()

        @pl.when(g + 2 < NGRP)
        def _fire():
            for dsc in coord_descs(g + 2, p):
                dsc.start()

        rbase = g * 16 + il
        rown = plsc.load_gather(rcn_v, [rbase, zero16]) * inv_b
        clumn = plsc.load_gather(rcn_v, [rbase, one16]) * inv_a
        srow = jnp.zeros((16,), jnp.float32)
        sclum = jnp.zeros((16,), jnp.float32)
        nb = il * S
        for j in range(S):
            srow = srow + plsc.load_gather(rcnb_v.at[p], [nb + j, zero16])
            sclum = sclum + plsc.load_gather(rcnb_v.at[p], [nb + j, one16])
        row_sum = srow * (1.0 / S) * inv_b
        clum_sum = sclum * (1.0 / S) * inv_a
        dr = row_sum - rown
        dc = clum_sum - clumn
        d2 = dr * dr + dc * dc + 1e-12
        dw = 1.0 / (1.0 + jnp.exp(-1.0 / d2))
        scale_v[pl.ds(g * 16, 16)] = dw * (1.0 / S)

    def dw_outer(t, carry):
        for p in range(2):
            dw_group(t * 2 + p, p)
        return carry

    with jax.named_scope("dweight"):
        lax.fori_loop(0, NGRP // 2, dw_outer, 0)

    semo = (semo0, semo1)

    def store_desc(c, po):
        s = base + c * CROWS
        return pltpu.make_async_copy(
            obuf.at[po], out.at[pl.ds(s, CROWS)], semo[po])

    def chunk_compute(gbuf, c, po):
        def row_body(r, carry):
            accs = [gbuf[r * S, pl.ds(k * 16, 16)] for k in range(D // 16)]
            for j in range(1, S):
                for k in range(D // 16):
                    accs[k] = accs[k] + gbuf[r * S + j, pl.ds(k * 16, 16)]
            widx = jnp.zeros((16,), jnp.int32) + (c * CROWS + r)
            w2 = 2.0 * plsc.load_gather(scale_v, [widx])
            for k in range(D // 16):
                e = jnp.exp(w2 * accs[k])
                obuf[po, r, pl.ds(k * 16, 16)] = (e - 1.0) / (e + 1.0)
            return carry

        lax.fori_loop(0, CROWS, row_body, 0)

    def outer(t, carry):
        for p in range(4):
            c = t * 4 + p
            po = p % 2
            feat_desc(c, p).wait()

            @pl.when(c + 3 < NCHUNK)
            def _fire():
                feat_desc(c + 3, (p + 3) % 4).start()

            @pl.when(base + (c - 2) * CROWS < BATCH)
            def _drain():
                @pl.when(c >= 2)
                def _():
                    store_desc(c - 2, po).wait()

            chunk_compute(gb[p], c, po)

            @pl.when(base + c * CROWS < BATCH)
            def _store():
                store_desc(c, po).start()
        return carry

    with jax.named_scope("mainloop"):
        lax.fori_loop(0, NCHUNK // 4, outer, 0)

    for c in (NCHUNK - 2, NCHUNK - 1):
        @pl.when(base + c * CROWS < BATCH)
        def _drain_tail(c=c):
            store_desc(c, c % 2).wait()


@jax.jit
def kernel(nodes, neigh_idx, features, row, clum):
    nodes_p = jnp.pad(nodes, (0, BP - BATCH)).reshape(BP // 40, 40)
    neigh_p = jnp.pad(neigh_idx.reshape(-1), (0, (BP - BATCH) * S))
    neigh2d = neigh_p.reshape(BP * S // 128, 128)
    rc = jnp.concatenate(
        [row[:, None], clum[:, None],
         jnp.zeros((N_NODES, 14), jnp.float32)], axis=1)

    mesh = plsc.VectorSubcoreMesh(core_axis_name="c", subcore_axis_name="s")
    f = functools.partial(
        pl.kernel,
        out_type=jax.ShapeDtypeStruct((BATCH, D), jnp.float32),
        mesh=mesh,
        compiler_params=pltpu.CompilerParams(
            needs_layout_passes=False, use_tc_tiling_on_sc=False),
        scratch_types=[
            pltpu.VMEM((8, 40), jnp.int32),
            pltpu.VMEM((NCHUNK, 128), jnp.int32),
            pltpu.VMEM((BPW, 16), jnp.float32),
            pltpu.VMEM((2, 256, 16), jnp.float32),
            pltpu.VMEM((BPW,), jnp.float32),
            pltpu.VMEM((RED,), jnp.float32),
            pltpu.VMEM((REDTAIL,), jnp.float32),
            pltpu.VMEM((2, 16), jnp.float32),
            pltpu.VMEM((NS, 2, 16), jnp.float32),
            pltpu.VMEM((2, CROWS, D), jnp.float32),
            pltpu.VMEM((128, D), jnp.float32),
            pltpu.VMEM((128, D), jnp.float32),
            pltpu.VMEM((128, D), jnp.float32),
            pltpu.VMEM((128, D), jnp.float32),
            pltpu.VMEM_SHARED((NS, 2, 16), jnp.float32),
            pltpu.SemaphoreType.DMA,
            pltpu.SemaphoreType.DMA,
            pltpu.SemaphoreType.DMA,
            pltpu.SemaphoreType.DMA,
            pltpu.SemaphoreType.DMA,
            pltpu.SemaphoreType.DMA,
            pltpu.SemaphoreType.DMA,
            pltpu.SemaphoreType.DMA,
            pltpu.SemaphoreType.DMA,
            pltpu.SemaphoreType.DMA,
        ],
    )(_sc_body)
    return f(nodes_p, neigh2d, features, rc, row, clum)


# speedup vs baseline: 3.0210x; 2.8431x over previous
import functools

import jax
import jax.numpy as jnp
from jax import lax
from jax.experimental import pallas as pl
from jax.experimental.pallas import tpu as pltpu
from jax.experimental.pallas import tpu_sc as plsc

N_NODES = 100000
BATCH = 10000
S = 16
D = 128

NC = 2
NS = 16
NW = NC * NS
BP = 10240
BPW = BP // NW
CROWS = 8
NCHUNK = BPW // CROWS
RED = 6240
REDTAIL = N_NODES - RED * NS
NGRP = BPW // 16


def _sc_body(nodes2d, neigh2d, feat, rc, rowv, clumv, out,
             nidx2, gidx2, rcn_v, rcnb_v, scale_v, redbuf, tailbuf,
             pbuf, allbuf, obuf, gbuf0, gbuf1, gbuf2, gbuf3, shared,
             semi, semn, semc0, semc1, semf0, semf1, semf2, semf3,
             semo0, semo1):
    cid = lax.axis_index("c")
    sid = lax.axis_index("s")
    wid = sid * NC + cid
    base = wid * BPW

    d_nidx = pltpu.make_async_copy(nodes2d.at[pl.ds(wid * 8, 8)], nidx2, semi)
    d_gidx = pltpu.make_async_copy(
        neigh2d.at[pl.ds(wid * NCHUNK, NCHUNK)], gidx2, semi)
    d_nidx.start()
    d_gidx.start()

    scope_maxred = jax.named_scope("maxred")
    scope_maxred.__enter__()
    neg = jnp.full((16,), -3.0e38, jnp.float32)

    def _reduce_slice(src):
        pltpu.sync_copy(src.at[pl.ds(sid * RED, RED)], redbuf)
        pltpu.sync_copy(src.at[pl.ds(NS * RED, REDTAIL)], tailbuf)

        def body(i, m):
            for k in range(10):
                m = jnp.maximum(m, redbuf[pl.ds(i * 160 + k * 16, 16)])
            return m

        m = lax.fori_loop(0, RED // 160, body, neg)
        for k in range(REDTAIL // 16):
            m = jnp.maximum(m, tailbuf[pl.ds(k * 16, 16)])
        return m

    mrow = _reduce_slice(rowv)
    mclum = _reduce_slice(clumv)

    pbuf[0, :] = mrow
    pbuf[1, :] = mclum
    pltpu.sync_copy(pbuf, shared.at[sid])
    plsc.subcore_barrier()
    pltpu.sync_copy(shared, allbuf)
    for t in range(NS):
        mrow = jnp.maximum(mrow, allbuf[t, 0, :])
        mclum = jnp.maximum(mclum, allbuf[t, 1, :])

    il = lax.iota(jnp.int32, 16)

    def _lane_max(v):
        for sh in (8, 4, 2, 1):
            tailbuf[pl.ds(0, 16)] = v
            v = jnp.maximum(v, plsc.load_gather(tailbuf, [il ^ sh]))
        return v

    inv_b = 1.0 / _lane_max(mrow)
    inv_a = 1.0 / _lane_max(mclum)
    scope_maxred.__exit__(None, None, None)

    scope_stage = jax.named_scope("idxwait")
    scope_stage.__enter__()
    d_nidx.wait()
    d_gidx.wait()
    scope_stage.__exit__(None, None, None)

    gb = (gbuf0, gbuf1, gbuf2, gbuf3)
    semf = (semf0, semf1, semf2, semf3)

    def feat_desc(c, p):
        return pltpu.make_async_copy(feat.at[gidx2.at[c]], gb[p], semf[p])

    def chunk_valid(c):
        return base + c * CROWS < BATCH

    for c in range(3):
        @pl.when(chunk_valid(c))
        def _prime_feat(c=c):
            feat_desc(c, c).start()

    ndescs = []
    for t in range(8):
        dsc = pltpu.make_async_copy(
            rc.at[nidx2.at[t]], rcn_v.at[pl.ds(t * 40, 40)], semn)
        dsc.start()
        ndescs.append(dsc)

    zero16 = jnp.zeros((16,), jnp.int32)
    one16 = zero16 + 1
    semc = (semc0, semc1)

    def coord_descs(g, p):
        return (
            pltpu.make_async_copy(
                rc.at[gidx2.at[2 * g]], rcnb_v.at[p, pl.ds(0, 128)], semc[p]),
            pltpu.make_async_copy(
                rc.at[gidx2.at[2 * g + 1]], rcnb_v.at[p, pl.ds(128, 128)],
                semc[p]),
        )

    def grp_valid(g):
        return base + g * 16 < BATCH

    for g in range(2):
        @pl.when(grp_valid(g))
        def _prime(g=g):
            for dsc in coord_descs(g, g):
                dsc.start()

    for dd in ndescs:
        dd.wait()

    def dw_group_inner(g, p):
        for dsc in coord_descs(g, p):
            dsc.wait()

        @pl.when(jnp.logical_and(g + 2 < NGRP, grp_valid(g + 2)))
        def _fire():
            for dsc in coord_descs(g + 2, p):
                dsc.start()

        rbase = g * 16 + il
        rown = plsc.load_gather(rcn_v, [rbase, zero16]) * inv_b
        clumn = plsc.load_gather(rcn_v, [rbase, one16]) * inv_a
        srow = jnp.zeros((16,), jnp.float32)
        sclum = jnp.zeros((16,), jnp.float32)
        nb = il * S
        for j in range(S):
            srow = srow + plsc.load_gather(rcnb_v.at[p], [nb + j, zero16])
            sclum = sclum + plsc.load_gather(rcnb_v.at[p], [nb + j, one16])
        row_sum = srow * (1.0 / S) * inv_b
        clum_sum = sclum * (1.0 / S) * inv_a
        dr = row_sum - rown
        dc = clum_sum - clumn
        d2 = dr * dr + dc * dc + 1e-12
        dw = 1.0 / (1.0 + jnp.exp(-1.0 / d2))
        scale_v[pl.ds(g * 16, 16)] = dw * (1.0 / S)

    def dw_group(g, p):
        @pl.when(grp_valid(g))
        def _():
            dw_group_inner(g, p)

    def dw_outer(t, carry):
        for p in range(2):
            dw_group(t * 2 + p, p)
        return carry

    with jax.named_scope("dweight"):
        lax.fori_loop(0, NGRP // 2, dw_outer, 0)

    semo = (semo0, semo1)

    def store_desc(c, po):
        s = base + c * CROWS
        return pltpu.make_async_copy(
            obuf.at[po], out.at[pl.ds(s, CROWS)], semo[po])

    def chunk_compute(gbuf, c, po):
        def row_body(r, carry):
            accs = [gbuf[r * S, pl.ds(k * 16, 16)] for k in range(D // 16)]
            for j in range(1, S):
                for k in range(D // 16):
                    accs[k] = accs[k] + gbuf[r * S + j, pl.ds(k * 16, 16)]
            widx = jnp.zeros((16,), jnp.int32) + (c * CROWS + r)
            w2 = 2.0 * plsc.load_gather(scale_v, [widx])
            for k in range(D // 16):
                e = jnp.exp(w2 * accs[k])
                obuf[po, r, pl.ds(k * 16, 16)] = (e - 1.0) / (e + 1.0)
            return carry

        lax.fori_loop(0, CROWS, row_body, 0)

    def outer(t, carry):
        for p in range(4):
            c = t * 4 + p
            po = p % 2

            @pl.when(chunk_valid(c))
            def _chunk():
                feat_desc(c, p).wait()

                @pl.when(jnp.logical_and(c + 3 < NCHUNK, chunk_valid(c + 3)))
                def _fire():
                    feat_desc(c + 3, (p + 3) % 4).start()

                @pl.when(c >= 2)
                def _drain():
                    store_desc(c - 2, po).wait()

                chunk_compute(gb[p], c, po)
                store_desc(c, po).start()
        return carry

    with jax.named_scope("mainloop"):
        lax.fori_loop(0, NCHUNK // 4, outer, 0)

    store_desc(0, 0).wait()
    store_desc(1, 1).wait()


@jax.jit
def kernel(nodes, neigh_idx, features, row, clum):
    nodes_p = jnp.pad(nodes, (0, BP - BATCH)).reshape(BP // 40, 40)
    neigh_p = jnp.pad(neigh_idx.reshape(-1), (0, (BP - BATCH) * S))
    neigh2d = neigh_p.reshape(BP * S // 128, 128)
    rc = jnp.concatenate(
        [row[:, None], clum[:, None],
         jnp.zeros((N_NODES, 14), jnp.float32)], axis=1)

    mesh = plsc.VectorSubcoreMesh(core_axis_name="c", subcore_axis_name="s")
    f = functools.partial(
        pl.kernel,
        out_type=jax.ShapeDtypeStruct((BATCH, D), jnp.float32),
        mesh=mesh,
        compiler_params=pltpu.CompilerParams(
            needs_layout_passes=False, use_tc_tiling_on_sc=False),
        scratch_types=[
            pltpu.VMEM((8, 40), jnp.int32),
            pltpu.VMEM((NCHUNK, 128), jnp.int32),
            pltpu.VMEM((BPW, 16), jnp.float32),
            pltpu.VMEM((2, 256, 16), jnp.float32),
            pltpu.VMEM((BPW,), jnp.float32),
            pltpu.VMEM((RED,), jnp.float32),
            pltpu.VMEM((REDTAIL,), jnp.float32),
            pltpu.VMEM((2, 16), jnp.float32),
            pltpu.VMEM((NS, 2, 16), jnp.float32),
            pltpu.VMEM((2, CROWS, D), jnp.float32),
            pltpu.VMEM((128, D), jnp.float32),
            pltpu.VMEM((128, D), jnp.float32),
            pltpu.VMEM((128, D), jnp.float32),
            pltpu.VMEM((128, D), jnp.float32),
            pltpu.VMEM_SHARED((NS, 2, 16), jnp.float32),
            pltpu.SemaphoreType.DMA,
            pltpu.SemaphoreType.DMA,
            pltpu.SemaphoreType.DMA,
            pltpu.SemaphoreType.DMA,
            pltpu.SemaphoreType.DMA,
            pltpu.SemaphoreType.DMA,
            pltpu.SemaphoreType.DMA,
            pltpu.SemaphoreType.DMA,
            pltpu.SemaphoreType.DMA,
            pltpu.SemaphoreType.DMA,
        ],
    )(_sc_body)
    return f(nodes_p, neigh2d, features, rc, row, clum)
